# baseline (speedup 1.0000x reference)
import jax
import jax.numpy as jnp
from jax.experimental import pallas as pl
from jax.experimental.pallas import tpu as pltpu

B = 16384
D_IN = 128
D_OUT = 128
BLOCK_ROWS = 16384
_Q = BLOCK_ROWS // 128


def _fused_body(ind_ref, x_ref, w_ref, b_ref, o_ref):
    i = pl.program_id(0)
    ind = ind_ref[pl.ds(i * _Q, _Q), :]
    y = jnp.dot(x_ref[...], w_ref[...], preferred_element_type=jnp.float32)
    y = y + b_ref[...]
    y3 = y.reshape(_Q, 128, D_OUT)
    masked = jnp.where(ind[:, :, None] != 0, y3, 0.0)
    o_ref[...] = masked.reshape(BLOCK_ROWS, D_OUT)


def kernel(indicators, data, W, b):
    ind2d = indicators.reshape(128, 128)
    b2d = b.reshape(1, D_OUT)
    grid = B // BLOCK_ROWS
    return pl.pallas_call(
        _fused_body,
        grid=(grid,),
        in_specs=[
            pl.BlockSpec((128, 128), lambda i: (0, 0)),
            pl.BlockSpec((BLOCK_ROWS, D_IN), lambda i: (i, 0)),
            pl.BlockSpec((D_IN, D_OUT), lambda i: (0, 0)),
            pl.BlockSpec((1, D_OUT), lambda i: (0, 0)),
        ],
        out_specs=pl.BlockSpec((BLOCK_ROWS, D_OUT), lambda i: (i, 0)),
        out_shape=jax.ShapeDtypeStruct((B, D_OUT), jnp.float32),
        compiler_params=pltpu.CompilerParams(
            dimension_semantics=("arbitrary",),
        ),
    )(ind2d, data, W, b2d)


# speedup vs baseline: 1.2935x; 1.2935x over previous
import jax
import jax.numpy as jnp
from jax.experimental import pallas as pl
from jax.experimental.pallas import tpu as pltpu

B = 16384
D_IN = 128
D_OUT = 128
BLOCK_ROWS = 8192
_Q = BLOCK_ROWS // 128


def _fused_body(ind_ref, x_ref, w_ref, b_ref, o_ref):
    i = pl.program_id(0)
    ind = ind_ref[pl.ds(i * _Q, _Q), :]
    y = jnp.dot(x_ref[...], w_ref[...], preferred_element_type=jnp.float32)
    y = y + b_ref[...]
    y3 = y.reshape(_Q, 128, D_OUT)
    masked = jnp.where(ind[:, :, None] != 0, y3, 0.0)
    o_ref[...] = masked.reshape(BLOCK_ROWS, D_OUT)


def kernel(indicators, data, W, b):
    ind2d = indicators.reshape(128, 128)
    b2d = b.reshape(1, D_OUT)
    grid = B // BLOCK_ROWS
    return pl.pallas_call(
        _fused_body,
        grid=(grid,),
        in_specs=[
            pl.BlockSpec((128, 128), lambda i: (0, 0)),
            pl.BlockSpec((BLOCK_ROWS, D_IN), lambda i: (i, 0)),
            pl.BlockSpec((D_IN, D_OUT), lambda i: (0, 0)),
            pl.BlockSpec((1, D_OUT), lambda i: (0, 0)),
        ],
        out_specs=pl.BlockSpec((BLOCK_ROWS, D_OUT), lambda i: (i, 0)),
        out_shape=jax.ShapeDtypeStruct((B, D_OUT), jnp.float32),
        compiler_params=pltpu.CompilerParams(
            dimension_semantics=("parallel",),
        ),
    )(ind2d, data, W, b2d)
